# Initial kernel scaffold; baseline (speedup 1.0000x reference)
import jax
import jax.numpy as jnp
from jax.experimental import pallas as pl


def kernel(x, Wr, W1, W2):
    raise NotImplementedError("write your pallas kernel here")


# speedup vs baseline: 1.0853x; 1.0853x over previous
import functools

import jax
import jax.numpy as jnp
from jax.experimental import pallas as pl
from jax.experimental.pallas import tpu as pltpu

DIM = 1024
HIDDEN = 4096
NUM_EXPERTS = 8
TOP_K = 2
HT = 1024


def _moe_body(xt_ref, wr_ref, w1_ref, w2_ref, out_ref):
    e = pl.program_id(0)
    h = pl.program_id(1)

    xt = xt_ref[...]

    logits = jnp.dot(wr_ref[...], xt, preferred_element_type=jnp.float32)
    m = jnp.max(logits, axis=0, keepdims=True)
    p = jnp.exp(logits - m)
    p = p / jnp.sum(p, axis=0, keepdims=True)

    iota = jax.lax.broadcasted_iota(jnp.int32, p.shape, 0)
    big = jnp.int32(NUM_EXPERTS)

    m1 = jnp.max(p, axis=0, keepdims=True)
    idx1 = jnp.min(jnp.where(p == m1, iota, big), axis=0, keepdims=True)
    mask1 = iota == idx1

    p2 = jnp.where(mask1, -1.0, p)
    m2 = jnp.max(p2, axis=0, keepdims=True)
    idx2 = jnp.min(jnp.where(p2 == m2, iota, big), axis=0, keepdims=True)
    mask2 = iota == idx2

    denom = m1 + m2
    coeff = jnp.where(mask1 | mask2, p, 0.0) / denom
    coeff_e = jnp.sum(jnp.where(iota == e, coeff, 0.0), axis=0, keepdims=True)

    h1 = jnp.dot(w1_ref[0], xt, preferred_element_type=jnp.float32)
    h1 = h1 * jax.nn.sigmoid(h1)
    part = jnp.dot(w2_ref[0], h1, preferred_element_type=jnp.float32)
    part = part * coeff_e

    @pl.when((e == 0) & (h == 0))
    def _init():
        out_ref[...] = part

    @pl.when((e > 0) | (h > 0))
    def _acc():
        out_ref[...] += part


@functools.partial(jax.jit, static_argnames=())
def kernel(x, Wr, W1, W2):
    b, s, d = x.shape
    n_tok = b * s
    xt = x.reshape(n_tok, d).T

    n_ht = HIDDEN // HT
    grid = (NUM_EXPERTS, n_ht)

    out_t = pl.pallas_call(
        _moe_body,
        grid=grid,
        in_specs=[
            pl.BlockSpec((d, n_tok), lambda e, h: (0, 0)),
            pl.BlockSpec((NUM_EXPERTS, d), lambda e, h: (0, 0)),
            pl.BlockSpec((1, HT, d), lambda e, h: (e, h, 0)),
            pl.BlockSpec((1, d, HT), lambda e, h: (e, 0, h)),
        ],
        out_specs=pl.BlockSpec((d, n_tok), lambda e, h: (0, 0)),
        out_shape=jax.ShapeDtypeStruct((d, n_tok), jnp.float32),
        compiler_params=pltpu.CompilerParams(
            dimension_semantics=("arbitrary", "arbitrary"),
        ),
    )(xt, Wr, W1, W2)

    return out_t.T.reshape(b, s, d)


# speedup vs baseline: 1.1892x; 1.0958x over previous
import functools

import jax
import jax.numpy as jnp
from jax.experimental import pallas as pl
from jax.experimental.pallas import tpu as pltpu

DIM = 1024
HIDDEN = 4096
NUM_EXPERTS = 8
TOP_K = 2
HT = 2048


def _moe_body(xt_ref, wr_ref, w1_ref, w2_ref, out_ref):
    e = pl.program_id(0)
    h = pl.program_id(1)

    xt = xt_ref[...]

    logits = jnp.dot(wr_ref[...], xt, preferred_element_type=jnp.float32)
    m = jnp.max(logits, axis=0, keepdims=True)
    p = jnp.exp(logits - m)
    p = p / jnp.sum(p, axis=0, keepdims=True)

    iota = jax.lax.broadcasted_iota(jnp.int32, p.shape, 0)
    big = jnp.int32(NUM_EXPERTS)

    m1 = jnp.max(p, axis=0, keepdims=True)
    idx1 = jnp.min(jnp.where(p == m1, iota, big), axis=0, keepdims=True)
    mask1 = iota == idx1

    p2 = jnp.where(mask1, -1.0, p)
    m2 = jnp.max(p2, axis=0, keepdims=True)
    idx2 = jnp.min(jnp.where(p2 == m2, iota, big), axis=0, keepdims=True)
    mask2 = iota == idx2

    denom = m1 + m2
    coeff = jnp.where(mask1 | mask2, p, 0.0) / denom
    coeff_e = jnp.sum(jnp.where(iota == e, coeff, 0.0), axis=0, keepdims=True)

    h1 = jnp.dot(w1_ref[0], xt, preferred_element_type=jnp.float32)
    h1 = h1 * jax.nn.sigmoid(h1)
    part = jnp.dot(w2_ref[0], h1, preferred_element_type=jnp.float32)
    part = part * coeff_e

    @pl.when((e == 0) & (h == 0))
    def _init():
        out_ref[...] = part

    @pl.when((e > 0) | (h > 0))
    def _acc():
        out_ref[...] += part


@functools.partial(jax.jit, static_argnames=())
def kernel(x, Wr, W1, W2):
    b, s, d = x.shape
    n_tok = b * s
    xt = x.reshape(n_tok, d).T

    n_ht = HIDDEN // HT
    grid = (NUM_EXPERTS, n_ht)

    out_t = pl.pallas_call(
        _moe_body,
        grid=grid,
        in_specs=[
            pl.BlockSpec((d, n_tok), lambda e, h: (0, 0)),
            pl.BlockSpec((NUM_EXPERTS, d), lambda e, h: (0, 0)),
            pl.BlockSpec((1, HT, d), lambda e, h: (e, h, 0)),
            pl.BlockSpec((1, d, HT), lambda e, h: (e, 0, h)),
        ],
        out_specs=pl.BlockSpec((d, n_tok), lambda e, h: (0, 0)),
        out_shape=jax.ShapeDtypeStruct((d, n_tok), jnp.float32),
        compiler_params=pltpu.CompilerParams(
            dimension_semantics=("arbitrary", "arbitrary"),
        ),
    )(xt, Wr, W1, W2)

    return out_t.T.reshape(b, s, d)


# speedup vs baseline: 1.1902x; 1.0008x over previous
import functools

import jax
import jax.numpy as jnp
from jax.experimental import pallas as pl
from jax.experimental.pallas import tpu as pltpu

DIM = 1024
HIDDEN = 4096
NUM_EXPERTS = 8
TOP_K = 2
HT = 2048
N_HT = HIDDEN // HT
N_TOK = 64


def _moe_body(x_ref, wr_ref, w1_ref, w2_ref, out_ref, xt_s, coeff_s, acc_s):
    e = pl.program_id(0)
    h = pl.program_id(1)

    @pl.when((e == 0) & (h == 0))
    def _prologue():
        xt = x_ref[...].T
        xt_s[...] = xt
        logits = jnp.dot(wr_ref[...], xt, preferred_element_type=jnp.float32)
        m = jnp.max(logits, axis=0, keepdims=True)
        p = jnp.exp(logits - m)
        p = p / jnp.sum(p, axis=0, keepdims=True)

        iota = jax.lax.broadcasted_iota(jnp.int32, p.shape, 0)
        big = jnp.int32(NUM_EXPERTS)
        m1 = jnp.max(p, axis=0, keepdims=True)
        idx1 = jnp.min(jnp.where(p == m1, iota, big), axis=0, keepdims=True)
        mask1 = iota == idx1
        p2 = jnp.where(mask1, -1.0, p)
        m2 = jnp.max(p2, axis=0, keepdims=True)
        idx2 = jnp.min(jnp.where(p2 == m2, iota, big), axis=0, keepdims=True)
        mask2 = iota == idx2
        coeff_s[...] = jnp.where(mask1 | mask2, p, 0.0) / (m1 + m2)

    xt = xt_s[...]
    coeff = coeff_s[...]
    iota_e = jax.lax.broadcasted_iota(jnp.int32, coeff.shape, 0)
    coeff_e = jnp.sum(jnp.where(iota_e == e, coeff, 0.0), axis=0, keepdims=True)

    h1 = jnp.dot(w1_ref[0], xt, preferred_element_type=jnp.float32)
    h1 = h1 * jax.nn.sigmoid(h1)
    part = jnp.dot(w2_ref[0], h1, preferred_element_type=jnp.float32) * coeff_e

    @pl.when((e == 0) & (h == 0))
    def _init():
        acc_s[...] = part

    @pl.when((e > 0) | (h > 0))
    def _acc():
        acc_s[...] += part

    @pl.when((e == NUM_EXPERTS - 1) & (h == N_HT - 1))
    def _epilogue():
        out_ref[...] = acc_s[...].T


@functools.partial(jax.jit, static_argnames=())
def kernel(x, Wr, W1, W2):
    b, s, d = x.shape
    n_tok = b * s
    x_flat = x.reshape(n_tok, d)

    out = pl.pallas_call(
        _moe_body,
        grid=(NUM_EXPERTS, N_HT),
        in_specs=[
            pl.BlockSpec((n_tok, d), lambda e, h: (0, 0)),
            pl.BlockSpec((NUM_EXPERTS, d), lambda e, h: (0, 0)),
            pl.BlockSpec((1, HT, d), lambda e, h: (e, h, 0)),
            pl.BlockSpec((1, d, HT), lambda e, h: (e, 0, h)),
        ],
        out_specs=pl.BlockSpec((n_tok, d), lambda e, h: (0, 0)),
        out_shape=jax.ShapeDtypeStruct((n_tok, d), jnp.float32),
        scratch_shapes=[
            pltpu.VMEM((d, n_tok), jnp.float32),
            pltpu.VMEM((NUM_EXPERTS, n_tok), jnp.float32),
            pltpu.VMEM((d, n_tok), jnp.float32),
        ],
        compiler_params=pltpu.CompilerParams(
            dimension_semantics=("arbitrary", "arbitrary"),
        ),
    )(x_flat, Wr, W1, W2)

    return out.reshape(b, s, d)
